# baseline (speedup 1.0000x reference)
import jax
import jax.numpy as jnp
from jax import lax
from jax.experimental import pallas as pl
from jax.experimental.pallas import tpu as pltpu
from jax.experimental.pallas import tpu_sc as plsc

N_NODES = 10000
D = 128
B = 320000

_INFO = plsc.get_sparse_core_info()
NC = _INFO.num_cores
NS = _INFO.num_subcores
L = _INFO.num_lanes

E_SC = B // NC
ECH = 2000
NCH = E_SC // ECH
NGRP = ECH // L
FPT = D // NS
ROWS_OUT = NCH // NS


def _body(zt_hbm, srci_hbm, dsti_hbm, cids_hbm, out_hbm,
          zts, is0, id0, is1, id1, pb0, pb1, cidsv, ostage,
          acc, isem0, isem1, psem0, psem1, osem):
    cid = lax.axis_index("c")
    sid = lax.axis_index("s")
    sc_base = cid * E_SC

    pltpu.sync_copy(zt_hbm.at[pl.ds(sid * FPT, FPT)], zts)
    pltpu.sync_copy(cids_hbm, cidsv)

    zero = jnp.zeros((L,), jnp.float32)

    def zrow(o, carry):
        pb0[0, pl.ds(o * L, L)] = zero
        return carry

    lax.fori_loop(0, NGRP, zrow, 0)
    for r in range(ROWS_OUT):
        pltpu.sync_copy(pb0.at[0], acc.at[sid * ROWS_OUT + r])

    plsc.subcore_barrier()

    zero16 = jnp.zeros((L,), jnp.int32)

    def issue_idx(c, ibs, ibd, isem):
        pltpu.async_copy(srci_hbm.at[pl.ds(sc_base + c * ECH, ECH)], ibs, isem)
        pltpu.async_copy(dsti_hbm.at[pl.ds(sc_base + c * ECH, ECH)], ibd, isem)

    def wait_idx(ibs, ibd, isem):
        pltpu.make_async_copy(srci_hbm.at[pl.ds(0, ECH)], ibs, isem).wait()
        pltpu.make_async_copy(srci_hbm.at[pl.ds(0, ECH)], ibd, isem).wait()

    def compute(ibs, ibd, pb):
        def group(g, carry):
            sidx = ibs[pl.ds(g * L, L)]
            didx = ibd[pl.ds(g * L, L)]
            acc_v = jnp.zeros((L,), jnp.float32)
            sa, da = sidx, didx
            for jj in range(FPT):
                if jj:
                    sa = sa + N_NODES
                    da = da + N_NODES
                sv = plsc.load_gather(zts, [zero16, sa])
                dv = plsc.load_gather(zts, [zero16, da])
                acc_v = acc_v + sv * dv
            pb[0, pl.ds(g * L, L)] = acc_v
            return carry

        lax.fori_loop(0, NGRP, group, 0)

    def issue_add(c, pb, psem):
        pltpu.async_copy(pb, acc.at[cidsv.at[c]], psem, add=True)

    def wait_add(pb, psem):
        pltpu.make_async_copy(pb, acc.at[cidsv.at[0]], psem).wait()

    issue_idx(0, is0, id0, isem0)
    issue_idx(1, is1, id1, isem1)

    def pair(i, carry):
        c0 = 2 * i
        c1 = 2 * i + 1

        wait_idx(is0, id0, isem0)

        @pl.when(i > 0)
        def _():
            wait_add(pb0, psem0)

        compute(is0, id0, pb0)
        issue_add(c0, pb0, psem0)

        @pl.when(c0 + 2 < NCH)
        def _():
            issue_idx(c0 + 2, is0, id0, isem0)

        wait_idx(is1, id1, isem1)

        @pl.when(i > 0)
        def _():
            wait_add(pb1, psem1)

        compute(is1, id1, pb1)
        issue_add(c1, pb1, psem1)

        @pl.when(c1 + 2 < NCH)
        def _():
            issue_idx(c1 + 2, is1, id1, isem1)

        return carry

    lax.fori_loop(0, NCH // 2, pair, 0)

    wait_add(pb0, psem0)
    wait_add(pb1, psem1)
    plsc.subcore_barrier()

    for r in range(ROWS_OUT):
        pltpu.sync_copy(acc.at[sid * ROWS_OUT + r], ostage)

        def sgrp(o, carry):
            v = ostage[pl.ds(o * L, L)]
            ostage[pl.ds(o * L, L)] = 1.0 / (1.0 + jnp.exp(-v))
            return carry

        lax.fori_loop(0, NGRP, sgrp, 0)
        pltpu.async_copy(
            ostage,
            out_hbm.at[pl.ds(sc_base + (sid * ROWS_OUT + r) * ECH, ECH)],
            osem)
        pltpu.make_async_copy(
            ostage, out_hbm.at[pl.ds(0, ECH)], osem).wait()


@jax.jit
def _run(zt, src, dst, cids):
    mesh = plsc.VectorSubcoreMesh(core_axis_name="c", subcore_axis_name="s")
    k = pl.kernel(
        _body,
        mesh=mesh,
        compiler_params=pltpu.CompilerParams(needs_layout_passes=False, use_tc_tiling_on_sc=False),
        out_type=jax.ShapeDtypeStruct((B,), jnp.float32),
        scratch_types=[
            pltpu.VMEM((FPT, N_NODES), jnp.float32),
            pltpu.VMEM((ECH,), jnp.int32),
            pltpu.VMEM((ECH,), jnp.int32),
            pltpu.VMEM((ECH,), jnp.int32),
            pltpu.VMEM((ECH,), jnp.int32),
            pltpu.VMEM((1, ECH), jnp.float32),
            pltpu.VMEM((1, ECH), jnp.float32),
            pltpu.VMEM((NCH, 1), jnp.int32),
            pltpu.VMEM((ECH,), jnp.float32),
            pltpu.VMEM_SHARED((NCH, ECH), jnp.float32),
            pltpu.SemaphoreType.DMA,
            pltpu.SemaphoreType.DMA,
            pltpu.SemaphoreType.DMA,
            pltpu.SemaphoreType.DMA,
            pltpu.SemaphoreType.DMA,
        ],
    )
    return k(zt, src, dst, cids)


def kernel(z, edge_index):
    zt = z.T
    src = edge_index[0].astype(jnp.int32)
    dst = edge_index[1].astype(jnp.int32)
    cids = jnp.arange(NCH, dtype=jnp.int32).reshape(NCH, 1)
    return _run(zt, src, dst, cids)


# speedup vs baseline: 1.1078x; 1.1078x over previous
import jax
import jax.numpy as jnp
from jax import lax
from jax.experimental import pallas as pl
from jax.experimental.pallas import tpu as pltpu
from jax.experimental.pallas import tpu_sc as plsc

N_NODES = 10000
D = 128
B = 320000

_INFO = plsc.get_sparse_core_info()
NC = _INFO.num_cores
NS = _INFO.num_subcores
NW = NC * NS
L = _INFO.num_lanes

EDGES_PER_W = B // NW
CHUNK = 64
N_CHUNKS = EDGES_PER_W // CHUNK
TAIL = EDGES_PER_W - N_CHUNKS * CHUNK
N_PAIRS = N_CHUNKS // 2
GROUPS = CHUNK // L
STRIPE = 640


def _body(z_hbm, srci_hbm, dsti_hbm, out_hbm,
          ib_s0, ib_d0, ib_s1, ib_d1, sb0, db0, sb1, db1, ob0, ob1, jvecs,
          zsh, sem0, sem1, isem0, isem1, osem0, osem1):
    cid = lax.axis_index("c")
    sid = lax.axis_index("s")
    wid = sid * NC + cid
    base = wid * EDGES_PER_W

    zoff = jnp.minimum(sid * STRIPE, N_NODES - STRIPE)
    pltpu.sync_copy(z_hbm.at[pl.ds(zoff, STRIPE)], zsh.at[pl.ds(zoff, STRIPE)])
    plsc.subcore_barrier()

    lanes = lax.iota(jnp.int32, L)

    lane_base = lanes * D

    def build_jvec(j, carry):
        jvecs[j] = lane_base + ((j + lanes) & (D - 1))
        return carry

    lax.fori_loop(0, D, build_jvec, 0)
    zero16 = jnp.zeros((L,), jnp.int32)

    def issue_idx(c, ib_s, ib_d, isem):
        pltpu.async_copy(srci_hbm.at[pl.ds(base + c * CHUNK, CHUNK)], ib_s, isem)
        pltpu.async_copy(dsti_hbm.at[pl.ds(base + c * CHUNK, CHUNK)], ib_d, isem)

    def wait_idx(ib_s, ib_d, isem):
        pltpu.make_async_copy(srci_hbm.at[pl.ds(0, CHUNK)], ib_s, isem).wait()
        pltpu.make_async_copy(dsti_hbm.at[pl.ds(0, CHUNK)], ib_d, isem).wait()

    def issue_rows(ib_s, ib_d, sb, db, sem):
        pltpu.async_copy(zsh.at[ib_s], sb, sem)
        pltpu.async_copy(zsh.at[ib_d], db, sem)

    def wait_rows(ib_s, ib_d, sb, db, sem):
        pltpu.make_async_copy(zsh.at[ib_s], sb, sem).wait()
        pltpu.make_async_copy(zsh.at[ib_d], db, sem).wait()

    def wait_out(ob, osem):
        pltpu.make_async_copy(ob, out_hbm.at[pl.ds(0, CHUNK)], osem).wait()

    def compute(c, sb, db, ob, osem, ngroups=GROUPS):
        zero = jnp.zeros((L,), jnp.float32)
        JBLK = 8

        def block(b, accs):
            accs = list(accs)
            for jj in range(JBLK):
                jv = jvecs[b * JBLK + jj]
                for g in range(ngroups):
                    fidx = (jv + (g * L * D)) if g else jv
                    sv = plsc.load_gather(sb, [zero16, fidx])
                    dv = plsc.load_gather(db, [zero16, fidx])
                    accs[g] = accs[g] + sv * dv
            return tuple(accs)

        accs = lax.fori_loop(0, D // JBLK, block, (zero,) * ngroups)
        for g in range(ngroups):
            res = 1.0 / (1.0 + jnp.exp(-accs[g]))
            ob[pl.ds(g * L, L)] = res
        pltpu.async_copy(ob.at[pl.ds(0, ngroups * L)],
                         out_hbm.at[pl.ds(base + c * CHUNK, ngroups * L)], osem)

    issue_idx(0, ib_s0, ib_d0, isem0)
    issue_idx(1, ib_s1, ib_d1, isem1)
    wait_idx(ib_s0, ib_d0, isem0)
    issue_rows(ib_s0, ib_d0, sb0, db0, sem0)

    def pair(i, carry):
        c0 = 2 * i
        c1 = 2 * i + 1

        wait_idx(ib_s1, ib_d1, isem1)
        issue_rows(ib_s1, ib_d1, sb1, db1, sem1)

        wait_rows(ib_s0, ib_d0, sb0, db0, sem0)

        @pl.when(c0 + 2 < N_CHUNKS)
        def _():
            issue_idx(c0 + 2, ib_s0, ib_d0, isem0)

        @pl.when(i > 0)
        def _():
            wait_out(ob0, osem0)

        compute(c0, sb0, db0, ob0, osem0)

        @pl.when(c0 + 2 < N_CHUNKS)
        def _():
            wait_idx(ib_s0, ib_d0, isem0)
            issue_rows(ib_s0, ib_d0, sb0, db0, sem0)

        wait_rows(ib_s1, ib_d1, sb1, db1, sem1)

        @pl.when(c1 + 2 < N_CHUNKS)
        def _():
            issue_idx(c1 + 2, ib_s1, ib_d1, isem1)

        @pl.when(i > 0)
        def _():
            wait_out(ob1, osem1)

        compute(c1, sb1, db1, ob1, osem1)
        return carry

    lax.fori_loop(0, N_PAIRS, pair, 0)

    pltpu.async_copy(srci_hbm.at[pl.ds(base + N_CHUNKS * CHUNK, TAIL)],
                     ib_s0.at[pl.ds(0, TAIL)], isem0).wait()
    pltpu.async_copy(dsti_hbm.at[pl.ds(base + N_CHUNKS * CHUNK, TAIL)],
                     ib_d0.at[pl.ds(0, TAIL)], isem0).wait()
    pltpu.async_copy(zsh.at[ib_s0.at[pl.ds(0, TAIL)]],
                     sb0.at[pl.ds(0, TAIL)], sem0).wait()
    pltpu.async_copy(zsh.at[ib_d0.at[pl.ds(0, TAIL)]],
                     db0.at[pl.ds(0, TAIL)], sem0).wait()
    wait_out(ob0, osem0)
    compute(N_CHUNKS, sb0, db0, ob0, osem0, ngroups=TAIL // L)
    wait_out(ob1, osem1)
    pltpu.make_async_copy(ob0.at[pl.ds(0, TAIL)],
                          out_hbm.at[pl.ds(0, TAIL)], osem0).wait()


@jax.jit
def _run(z, src, dst):
    mesh = plsc.VectorSubcoreMesh(core_axis_name="c", subcore_axis_name="s")
    k = pl.kernel(
        _body,
        mesh=mesh,
        compiler_params=pltpu.CompilerParams(needs_layout_passes=False),
        out_type=jax.ShapeDtypeStruct((B,), jnp.float32),
        scratch_types=[
            pltpu.VMEM((CHUNK,), jnp.int32),
            pltpu.VMEM((CHUNK,), jnp.int32),
            pltpu.VMEM((CHUNK,), jnp.int32),
            pltpu.VMEM((CHUNK,), jnp.int32),
            pltpu.VMEM((CHUNK, D), jnp.float32),
            pltpu.VMEM((CHUNK, D), jnp.float32),
            pltpu.VMEM((CHUNK, D), jnp.float32),
            pltpu.VMEM((CHUNK, D), jnp.float32),
            pltpu.VMEM((CHUNK,), jnp.float32),
            pltpu.VMEM((CHUNK,), jnp.float32),
            pltpu.VMEM((D, L), jnp.int32),
            pltpu.VMEM_SHARED((N_NODES, D), jnp.float32),
            pltpu.SemaphoreType.DMA,
            pltpu.SemaphoreType.DMA,
            pltpu.SemaphoreType.DMA,
            pltpu.SemaphoreType.DMA,
            pltpu.SemaphoreType.DMA,
            pltpu.SemaphoreType.DMA,
        ],
    )
    return k(z, src, dst)


def kernel(z, edge_index):
    src = edge_index[0].astype(jnp.int32)
    dst = edge_index[1].astype(jnp.int32)
    return _run(z, src, dst)
